# baseline (speedup 1.0000x reference)
import functools

import jax
import jax.numpy as jnp
from jax import lax
from jax.experimental import pallas as pl
from jax.experimental.pallas import tpu as pltpu
from jax.experimental.pallas import tpu_sc as plsc

B, S, D = 4096, 200, 64
TAB_ROWS = S + 1
NC, NS = 2, 16
NW = NC * NS
BW = B // NW
CS = 2
NCHUNK = S // CS
NIN = 5
NOUT = 2
GRP = 10
NGRP = NCHUNK // GRP
L = 16
NG = BW // L


@functools.partial(
    pl.kernel,
    mesh=plsc.VectorSubcoreMesh(core_axis_name="c", subcore_axis_name="s"),
    out_type=jax.ShapeDtypeStruct((S, D, B), jnp.float32),
    scratch_types=(
        [pltpu.VMEM((CS, D, BW), jnp.float32)] * NIN
        + [pltpu.VMEM((CS, D, BW), jnp.float32)] * NOUT
        + [pltpu.VMEM((CS, BW), jnp.float32)] * NIN
        + [pltpu.VMEM((TAB_ROWS * D,), jnp.float32)]
        + [pltpu.SemaphoreType.DMA] * (NIN + NOUT)
    ),
)
def _pos_enc_sc(x_hbm, mask_hbm, tab_hbm, out_hbm,
                xin0, xin1, xin2, xin3, xin4, xout0, xout1,
                mb0, mb1, mb2, mb3, mb4, tbuf,
                isem0, isem1, isem2, isem3, isem4, osem0, osem1):
    wid = lax.axis_index("s") * NC + lax.axis_index("c")
    b0 = wid * BW
    xin = (xin0, xin1, xin2, xin3, xin4)
    xout = (xout0, xout1)
    mb = (mb0, mb1, mb2, mb3, mb4)
    isem = (isem0, isem1, isem2, isem3, isem4)
    osem = (osem0, osem1)

    pltpu.sync_copy(tab_hbm, tbuf)

    def start_in(c, i):
        pltpu.async_copy(x_hbm.at[pl.ds(c * CS, CS), :, pl.ds(b0, BW)],
                         xin[i], isem[i])
        pltpu.async_copy(mask_hbm.at[pl.ds(c * CS, CS), pl.ds(b0, BW)],
                         mb[i], isem[i])

    def wait_in(i):
        pltpu.make_async_copy(x_hbm.at[pl.ds(0, CS), :, pl.ds(b0, BW)],
                              xin[i], isem[i]).wait()
        pltpu.make_async_copy(mask_hbm.at[pl.ds(0, CS), pl.ds(b0, BW)],
                              mb[i], isem[i]).wait()

    def start_out(c, i):
        pltpu.async_copy(xout[i], out_hbm.at[pl.ds(c * CS, CS), :, pl.ds(b0, BW)],
                         osem[i])

    def wait_out(i):
        pltpu.make_async_copy(xout[i], out_hbm.at[pl.ds(0, CS), :, pl.ds(b0, BW)],
                              osem[i]).wait()

    def compute(c, i, o):
        src, dst, msk = xin[i], xout[o], mb[i]
        s0 = c * CS

        @plsc.parallel_loop(0, CS, unroll=1)
        def s_body(sl):
            keep = [1.0 - msk[sl, pl.ds(g * L, L)] for g in range(NG)]
            row = (s0 + sl + 1) * D
            for dblk in range(D // L):
                tv = tbuf[pl.ds(row + dblk * L, L)]
                for j in range(L):
                    d = dblk * L + j
                    bval = jnp.broadcast_to(tv[j], (L,))
                    for g in range(NG):
                        dst[sl, d, pl.ds(g * L, L)] = (
                            src[sl, d, pl.ds(g * L, L)] + bval * keep[g]
                        )

    start_in(0, 0)
    start_in(1, 1)
    start_in(2, 2)
    start_in(3, 3)

    def group_body(g, carry):
        cbase = g * GRP
        for k in range(GRP):
            c = cbase + k
            i, o = k % NIN, k % NOUT
            wait_in(i)

            @pl.when(c >= NOUT)
            def _():
                wait_out(o)

            @pl.when(c + 4 < NCHUNK)
            def _():
                start_in(c + 4, (k + 4) % NIN)

            compute(c, i, o)
            start_out(c, o)
        return carry

    lax.fori_loop(0, NGRP, group_body, 0)
    wait_out(0)
    wait_out(1)


def kernel(x, key_padding_mask, pos_table):
    x_t = jnp.transpose(x, (1, 2, 0))
    mask_f = key_padding_mask.T.astype(jnp.float32)
    tab_flat = pos_table.reshape(TAB_ROWS * D)
    out_t = _pos_enc_sc(x_t, mask_f, tab_flat)
    return jnp.transpose(out_t, (2, 0, 1))


# speedup vs baseline: 1.0588x; 1.0588x over previous
import functools

import jax
import jax.numpy as jnp
from jax import lax
from jax.experimental import pallas as pl
from jax.experimental.pallas import tpu as pltpu
from jax.experimental.pallas import tpu_sc as plsc

B, S, D = 4096, 200, 64
TAB_ROWS = S + 1
NC, NS = 2, 16
NW = NC * NS
BW = B // NW
CS = 2
NCHUNK = S // CS
NIN = 4
NOUT = 2
GRP = 4
NGRP = NCHUNK // GRP
L = 16
NG = BW // L


@functools.partial(
    pl.kernel,
    mesh=plsc.VectorSubcoreMesh(core_axis_name="c", subcore_axis_name="s"),
    out_type=jax.ShapeDtypeStruct((S, D, B), jnp.float32),
    scratch_types=(
        [pltpu.VMEM((CS, D, BW), jnp.float32)] * NIN
        + [pltpu.VMEM((CS, D, BW), jnp.float32)] * NOUT
        + [pltpu.VMEM((CS, BW), jnp.float32)] * NIN
        + [pltpu.VMEM((TAB_ROWS * D,), jnp.float32)]
        + [pltpu.SemaphoreType.DMA] * (NIN + NOUT)
    ),
)
def _pos_enc_sc(x_hbm, mask_hbm, tab_hbm, out_hbm,
                xin0, xin1, xin2, xin3, xout0, xout1,
                mb0, mb1, mb2, mb3, tbuf,
                isem0, isem1, isem2, isem3, osem0, osem1):
    wid = lax.axis_index("s") * NC + lax.axis_index("c")
    b0 = wid * BW
    xin = (xin0, xin1, xin2, xin3)
    xout = (xout0, xout1)
    mb = (mb0, mb1, mb2, mb3)
    isem = (isem0, isem1, isem2, isem3)
    osem = (osem0, osem1)

    pltpu.sync_copy(tab_hbm, tbuf)

    def start_in(c, i):
        pltpu.async_copy(x_hbm.at[pl.ds(c * CS, CS), :, pl.ds(b0, BW)],
                         xin[i], isem[i])
        pltpu.async_copy(mask_hbm.at[pl.ds(c * CS, CS), pl.ds(b0, BW)],
                         mb[i], isem[i])

    def wait_in(i):
        pltpu.make_async_copy(x_hbm.at[pl.ds(0, CS), :, pl.ds(b0, BW)],
                              xin[i], isem[i]).wait()
        pltpu.make_async_copy(mask_hbm.at[pl.ds(0, CS), pl.ds(b0, BW)],
                              mb[i], isem[i]).wait()

    def start_out(c, i):
        pltpu.async_copy(xout[i], out_hbm.at[pl.ds(c * CS, CS), :, pl.ds(b0, BW)],
                         osem[i])

    def wait_out(i):
        pltpu.make_async_copy(xout[i], out_hbm.at[pl.ds(0, CS), :, pl.ds(b0, BW)],
                              osem[i]).wait()

    def compute(c, i, o):
        src, dst, msk = xin[i], xout[o], mb[i]
        s0 = c * CS

        @plsc.parallel_loop(0, CS, unroll=1)
        def s_body(sl):
            keep = [1.0 - msk[sl, pl.ds(g * L, L)] for g in range(NG)]
            row = (s0 + sl + 1) * D
            for dblk in range(D // L):
                tv = tbuf[pl.ds(row + dblk * L, L)]
                for j in range(L):
                    d = dblk * L + j
                    bval = jnp.broadcast_to(tv[j], (L,))
                    for g in range(NG):
                        dst[sl, d, pl.ds(g * L, L)] = (
                            src[sl, d, pl.ds(g * L, L)] + bval * keep[g]
                        )

    start_in(0, 0)
    start_in(1, 1)
    start_in(2, 2)

    def group_body(g, carry):
        cbase = g * GRP
        for k in range(GRP):
            c = cbase + k
            i, o = k % NIN, k % NOUT
            wait_in(i)

            @pl.when(c >= NOUT)
            def _():
                wait_out(o)

            @pl.when(c + 3 < NCHUNK)
            def _():
                start_in(c + 3, (k + 3) % NIN)

            compute(c, i, o)
            start_out(c, o)
        return carry

    lax.fori_loop(0, NGRP, group_body, 0)
    wait_out(0)
    wait_out(1)


def kernel(x, key_padding_mask, pos_table):
    x_t = jnp.transpose(x, (1, 2, 0))
    mask_f = key_padding_mask.T.astype(jnp.float32)
    tab_flat = pos_table.reshape(TAB_ROWS * D)
    out_t = _pos_enc_sc(x_t, mask_f, tab_flat)
    return jnp.transpose(out_t, (2, 0, 1))
